# baseline (speedup 1.0000x reference)
import jax
import jax.numpy as jnp
from jax import lax
from jax.experimental import pallas as pl
from jax.experimental.pallas import tpu as pltpu
from jax.experimental.pallas import tpu_sc as plsc

VOCAB = 32000
QVOCAB = 64
DYN_VOCAB = VOCAB + QVOCAB + 256
BT = 4096
EPS = 1e-20
LN2 = 0.6931471805599453
SQRT2 = 1.41421356

_INFO = plsc.get_sparse_core_info()
NC, NS, L = _INFO.num_cores, _INFO.num_subcores, _INFO.num_lanes
NW = NC * NS
ROWS = BT // NW
CH = ROWS // L


def _neg_log(p):
    bits = lax.bitcast_convert_type(p, jnp.int32)
    e = lax.shift_right_arithmetic(bits, 23) - 127
    m = lax.bitcast_convert_type((bits & 0x7FFFFF) | 0x3F800000, jnp.float32)
    big = m > SQRT2
    m = jnp.where(big, m * 0.5, m)
    ef = jnp.where(big, e + 1, e).astype(jnp.float32)
    z = (m - 1.0) / (m + 1.0)
    z2 = z * z
    poly = 2.0 + z2 * (2.0 / 3.0 + z2 * (0.4 + z2 * (2.0 / 7.0 + z2 * (2.0 / 9.0))))
    return -(ef * LN2 + z * poly)


def _loss_body(scores_hbm, tgt_hbm, aq_hbm, as_hbm, out_hbm,
               tgt_v, aq_v, as_v, it_v, is_v, cr_v, vt_v, vs_v, loss_v, sem):
    wid = lax.axis_index("s") * NC + lax.axis_index("c")
    base = wid * ROWS

    c1 = pltpu.async_copy(tgt_hbm.at[pl.ds(base, ROWS)], tgt_v, sem)
    c2 = pltpu.async_copy(aq_hbm.at[pl.ds(base, ROWS)], aq_v, sem)
    c3 = pltpu.async_copy(as_hbm.at[pl.ds(base, ROWS)], as_v, sem)
    c4 = pltpu.async_copy(
        scores_hbm.at[pl.ds(VOCAB, QVOCAB), pl.ds(base, ROWS)], cr_v, sem)
    c1.wait()
    for j in range(CH):
        sl = pl.ds(j * L, L)
        it_v[sl] = tgt_v[sl]
    gt = pltpu.async_copy(scores_hbm.at[it_v, pl.ds(base, ROWS)], vt_v, sem)
    c3.wait()
    for j in range(CH):
        sl = pl.ds(j * L, L)
        is_v[sl] = VOCAB + QVOCAB + as_v[sl]
    gs = pltpu.async_copy(scores_hbm.at[is_v, pl.ds(base, ROWS)], vs_v, sem)
    c2.wait()
    c4.wait()
    gt.wait()
    gs.wait()

    for j in range(CH):
        sl = pl.ds(j * L, L)
        lane = j * L + lax.iota(jnp.int32, L)
        t = tgt_v[sl]
        aq = aq_v[sl]
        asrc = as_v[sl]
        vp = plsc.load_gather(vt_v, [lane, lane])
        qp0 = plsc.load_gather(cr_v, [aq, lane])
        sp0 = plsc.load_gather(vs_v, [lane, lane])
        qp = jnp.where(aq == 0, 0.0, qp0) + EPS
        sp = jnp.where(asrc == 0, 0.0, sp0) + EPS
        non_copy = ((aq == 0) & (asrc == 0)) | (t != 0)
        probs = qp + sp + jnp.where(non_copy, vp, 0.0)
        loss = _neg_log(probs)
        loss_v[sl] = jnp.where(t == -100, 0.0, loss)

    pltpu.sync_copy(loss_v, out_hbm.at[pl.ds(base, ROWS)])


@jax.jit
def _run(scores, align_qry, align_src, target):
    scores_t = scores.T
    k = pl.kernel(
        _loss_body,
        out_type=jax.ShapeDtypeStruct((BT,), jnp.float32),
        mesh=plsc.VectorSubcoreMesh(core_axis_name="c", subcore_axis_name="s"),
        compiler_params=pltpu.CompilerParams(
            use_tc_tiling_on_sc=True, needs_layout_passes=False),
        scratch_types=[
            pltpu.VMEM((ROWS,), jnp.int32),
            pltpu.VMEM((ROWS,), jnp.int32),
            pltpu.VMEM((ROWS,), jnp.int32),
            pltpu.VMEM((ROWS,), jnp.int32),
            pltpu.VMEM((ROWS,), jnp.int32),
            pltpu.VMEM((QVOCAB, ROWS), jnp.float32),
            pltpu.VMEM((ROWS, ROWS), jnp.float32),
            pltpu.VMEM((ROWS, ROWS), jnp.float32),
            pltpu.VMEM((ROWS,), jnp.float32),
            pltpu.SemaphoreType.DMA,
        ],
    )
    return k(scores_t, target, align_qry, align_src)


def kernel(scores, qry_map, align_qry, align_src, target):
    del qry_map
    return _run(scores, align_qry, align_src, target)


# speedup vs baseline: 1.0370x; 1.0370x over previous
import jax
import jax.numpy as jnp
from jax import lax
from jax.experimental import pallas as pl
from jax.experimental.pallas import tpu as pltpu
from jax.experimental.pallas import tpu_sc as plsc

VOCAB = 32000
QVOCAB = 64
DYN_VOCAB = VOCAB + QVOCAB + 256
BT = 4096
EPS = 1e-20
LN2 = 0.6931471805599453
SQRT2 = 1.41421356

_INFO = plsc.get_sparse_core_info()
NC, NS, L = _INFO.num_cores, _INFO.num_subcores, _INFO.num_lanes
NW = NC * NS
ROWS = BT // NW
CH = ROWS // L


def _neg_log(p):
    bits = lax.bitcast_convert_type(p, jnp.int32)
    e = lax.shift_right_arithmetic(bits, 23) - 127
    m = lax.bitcast_convert_type((bits & 0x7FFFFF) | 0x3F800000, jnp.float32)
    big = m > SQRT2
    m = jnp.where(big, m * 0.5, m)
    ef = jnp.where(big, e + 1, e).astype(jnp.float32)
    z = (m - 1.0) / (m + 1.0)
    z2 = z * z
    poly = 2.0 + z2 * (2.0 / 3.0 + z2 * (0.4 + z2 * (2.0 / 7.0 + z2 * (2.0 / 9.0))))
    return -(ef * LN2 + z * poly)


def _loss_body(scores_hbm, tgt_hbm, aq_hbm, as_hbm, out_hbm,
               tgt_v, aq_v, as_v, it_v, cr_v, vt_v, loss_v, sem, sem_t):
    wid = lax.axis_index("s") * NC + lax.axis_index("c")
    base = wid * ROWS

    c1 = pltpu.async_copy(tgt_hbm.at[pl.ds(base, ROWS)], tgt_v, sem_t)
    c2 = pltpu.async_copy(aq_hbm.at[pl.ds(base, ROWS)], aq_v, sem)
    c3 = pltpu.async_copy(as_hbm.at[pl.ds(base, ROWS)], as_v, sem)
    c4 = pltpu.async_copy(
        scores_hbm.at[pl.ds(VOCAB, DYN_VOCAB - VOCAB), pl.ds(base, ROWS)],
        cr_v, sem)
    c1.wait()
    for j in range(CH):
        sl = pl.ds(j * L, L)
        it_v[sl] = tgt_v[sl]
    g = pltpu.async_copy(scores_hbm.at[it_v, pl.ds(base, ROWS)], vt_v, sem)
    c2.wait()
    c3.wait()
    c4.wait()
    g.wait()

    for j in range(CH):
        sl = pl.ds(j * L, L)
        lane = j * L + lax.iota(jnp.int32, L)
        t = tgt_v[sl]
        aq = aq_v[sl]
        asrc = as_v[sl]
        vp = plsc.load_gather(vt_v, [lane, lane])
        qp0 = plsc.load_gather(cr_v, [aq, lane])
        sp0 = plsc.load_gather(cr_v, [QVOCAB + asrc, lane])
        qp = jnp.where(aq == 0, 0.0, qp0) + EPS
        sp = jnp.where(asrc == 0, 0.0, sp0) + EPS
        non_copy = ((aq == 0) & (asrc == 0)) | (t != 0)
        probs = qp + sp + jnp.where(non_copy, vp, 0.0)
        loss = _neg_log(probs)
        loss_v[sl] = jnp.where(t == -100, 0.0, loss)

    pltpu.sync_copy(loss_v, out_hbm.at[pl.ds(base, ROWS)])


@jax.jit
def _run(scores, align_qry, align_src, target):
    scores_t = scores.T
    k = pl.kernel(
        _loss_body,
        out_type=jax.ShapeDtypeStruct((BT,), jnp.float32),
        mesh=plsc.VectorSubcoreMesh(core_axis_name="c", subcore_axis_name="s"),
        compiler_params=pltpu.CompilerParams(
            use_tc_tiling_on_sc=True, needs_layout_passes=False),
        scratch_types=[
            pltpu.VMEM((ROWS,), jnp.int32),
            pltpu.VMEM((ROWS,), jnp.int32),
            pltpu.VMEM((ROWS,), jnp.int32),
            pltpu.VMEM((ROWS,), jnp.int32),
            pltpu.VMEM((DYN_VOCAB - VOCAB, ROWS), jnp.float32),
            pltpu.VMEM((ROWS, ROWS), jnp.float32),
            pltpu.VMEM((ROWS,), jnp.float32),
            pltpu.SemaphoreType.DMA,
            pltpu.SemaphoreType.DMA,
        ],
    )
    return k(scores_t, target, align_qry, align_src)


def kernel(scores, qry_map, align_qry, align_src, target):
    del qry_map
    return _run(scores, align_qry, align_src, target)
